# baseline (speedup 1.0000x reference)
import functools

import jax
import jax.numpy as jnp
from jax import lax
from jax.experimental import pallas as pl
from jax.experimental.pallas import tpu as pltpu
from jax.experimental.pallas import tpu_sc as plsc

N = 10000
E = 320000
F_IN = 128
H = 8
F_OUT = 16
D = H * F_OUT
ROW = 144

NC = 2
NS = 16
NW = NC * NS
EPT = E // NW
B = 40
NB = EPT // B
N_PAD = 10240
RPT = N_PAD // NS


def _tc1_body(x_ref, w_ref, p_ref, h_ref, ad_ref):
    h = jnp.dot(x_ref[...], w_ref[...], preferred_element_type=jnp.float32)
    h_ref[...] = h
    ad_ref[...] = jnp.dot(h, p_ref[...], preferred_element_type=jnp.float32)


def _tc1(x, W, P16):
    blk = 1000
    grid = N // blk
    return pl.pallas_call(
        _tc1_body,
        grid=(grid,),
        in_specs=[
            pl.BlockSpec((blk, F_IN), lambda i: (i, 0)),
            pl.BlockSpec((F_IN, D), lambda i: (0, 0)),
            pl.BlockSpec((F_IN, 2 * H), lambda i: (0, 0)),
        ],
        out_specs=[
            pl.BlockSpec((blk, D), lambda i: (i, 0)),
            pl.BlockSpec((blk, 2 * H), lambda i: (i, 0)),
        ],
        out_shape=[
            jax.ShapeDtypeStruct((N, D), jnp.float32),
            jax.ShapeDtypeStruct((N, 2 * H), jnp.float32),
        ],
    )(x, W, P16)


def _sc_edge_pass(il, h, ad16, zeros):
    mesh = plsc.VectorSubcoreMesh(core_axis_name="c", subcore_axis_name="s",
                                  num_cores=NC, num_subcores=NS)

    @functools.partial(
        pl.kernel,
        out_type=jax.ShapeDtypeStruct((NC, N_PAD, ROW), jnp.float32),
        mesh=mesh,
        scratch_types=[
            pltpu.VMEM((2 * B,), jnp.int32),
            pltpu.VMEM((2 * B,), jnp.int32),
            pltpu.VMEM((B,), jnp.int32),
            pltpu.VMEM((B,), jnp.int32),
            pltpu.VMEM((2 * B, 2 * H), jnp.float32),
            pltpu.VMEM((2 * B, 2 * H), jnp.float32),
            pltpu.VMEM((B, D), jnp.float32),
            pltpu.VMEM((B, D), jnp.float32),
            pltpu.VMEM((B, ROW), jnp.float32),
            pltpu.VMEM((B, ROW), jnp.float32),
            pltpu.VMEM_SHARED((N_PAD, ROW), jnp.float32),
            pltpu.SemaphoreType.DMA,
            pltpu.SemaphoreType.DMA,
            pltpu.SemaphoreType.DMA,
            pltpu.SemaphoreType.DMA,
            pltpu.SemaphoreType.DMA,
            pltpu.SemaphoreType.DMA,
            pltpu.SemaphoreType.DMA,
            pltpu.SemaphoreType.DMA,
        ],
        compiler_params=pltpu.CompilerParams(
            use_tc_tiling_on_sc=False, needs_layout_passes=False),
    )
    def sc_kernel(il_hbm, h_hbm, ad_hbm, z_hbm, out_hbm,
                  sv0, sv1, dsc0, dsc1, g12_0, g12_1,
                  hr_0, hr_1, cb_0, cb_1, acc,
                  si0, si1, sd0, sd1, sg0, sg1, ss0, ss1):
        c = lax.axis_index("c")
        s = lax.axis_index("s")
        wid = c * NS + s
        bufs = ((sv0, dsc0, g12_0, hr_0, cb_0, si0, sd0, sg0, ss0),
                (sv1, dsc1, g12_1, hr_1, cb_1, si1, sd1, sg1, ss1))

        pltpu.sync_copy(z_hbm.at[pl.ds(s * RPT, RPT)],
                        acc.at[pl.ds(s * RPT, RPT)])
        zero16 = jnp.zeros((16,), jnp.float32)
        for e in range(B):
            cb_0[e, pl.ds(D, 16)] = zero16
            cb_1[e, pl.ds(D, 16)] = zero16
        plsc.subcore_barrier()

        base = wid * NB
        iota = lax.iota(jnp.int32, 16)
        rowpat = iota // 8
        colpat = iota % 8

        def issue_idx(i, p):
            sv, si = bufs[p][0], bufs[p][5]
            pltpu.async_copy(il_hbm.at[base + i], sv, si)

        def wait_idx(p):
            sv, si = bufs[p][0], bufs[p][5]
            pltpu.make_async_copy(il_hbm.at[0], sv, si).wait()

        def issue_g(p):
            sv, g12, hr = bufs[p][0], bufs[p][2], bufs[p][3]
            sg = bufs[p][7]
            pltpu.async_copy(ad_hbm.at[sv], g12, sg)
            pltpu.async_copy(h_hbm.at[sv.at[pl.ds(0, B)]], hr, sg)

        def drain_g(p):
            g12, hr = bufs[p][2], bufs[p][3]
            sg = bufs[p][7]
            pltpu.make_async_copy(ad_hbm.at[pl.ds(0, 2 * B)], g12, sg).wait()
            pltpu.make_async_copy(h_hbm.at[pl.ds(0, B)], hr, sg).wait()

        def issue_dsc(i, p):
            dsc, sd = bufs[p][1], bufs[p][6]
            pltpu.async_copy(il_hbm.at[base + i, pl.ds(B, B)], dsc, sd)

        def wait_dsc(p):
            dsc, sd = bufs[p][1], bufs[p][6]
            pltpu.make_async_copy(il_hbm.at[0, pl.ds(B, B)], dsc, sd).wait()

        def drain_s(p):
            cb, ss = bufs[p][4], bufs[p][8]
            pltpu.make_async_copy(out_hbm.at[0, pl.ds(0, B)], cb, ss).wait()

        def compute(p):
            g12, hr, cb = bufs[p][2], bufs[p][3], bufs[p][4]

            @plsc.parallel_loop(0, B // 2, 1, unroll=2)
            def pair(j):
                r = rowpat + 2 * j
                t = (plsc.load_gather(g12, [r, colpat])
                     + plsc.load_gather(g12, [r + B, colpat + 8]))
                w = jnp.exp(jnp.maximum(t, 0.2 * t))
                plsc.store_scatter(cb, [r, colpat + D], w)
                for k in range(16):
                    e = 2 * j + (k // 8)
                    col = (k % 8) * 16
                    bk = jnp.take_along_axis(
                        w, jnp.full((16,), k, jnp.int32), axis=0,
                        mode="promise_in_bounds")
                    cb[e, pl.ds(col, 16)] = hr[e, pl.ds(col, 16)] * bk

        def scatter(p):
            dsc, cb, ss = bufs[p][1], bufs[p][4], bufs[p][8]
            pltpu.async_copy(cb, acc.at[dsc], ss, add=True)

        issue_idx(0, 0)
        issue_idx(1, 1)
        wait_idx(0)
        issue_g(0)

        def pairbody(k, carry):
            for b in range(2):
                i = 2 * k + b

                def _nxt(p=b):
                    wait_idx(1 - p)
                    issue_g(1 - p)
                pl.when(i + 1 < NB)(_nxt)
                drain_g(b)

                def _pf(p=b, iv=i):
                    issue_idx(iv + 2, p)
                pl.when(i + 2 < NB)(_pf)

                pl.when(i >= 2)(lambda p=b: drain_s(p))
                issue_dsc(i, b)
                compute(b)
                wait_dsc(b)
                scatter(b)
            return carry

        lax.fori_loop(0, NB // 2, pairbody, 0)
        drain_s(0)
        drain_s(1)
        plsc.subcore_barrier()
        pltpu.sync_copy(acc.at[pl.ds(s * RPT, RPT)],
                        out_hbm.at[c, pl.ds(s * RPT, RPT)])

    return sc_kernel(il, h, ad16, zeros)


def _tc2_body(p0_ref, p1_ref, h_ref, ps_ref, q_ref, r_ref, b_ref, g_ref,
              be_ref, out_ref):
    S = p0_ref[...] + p1_ref[...]
    h = h_ref[...]
    msg = S[:, :D]
    t = jnp.dot(h, ps_ref[...], preferred_element_type=jnp.float32)
    wself = jnp.exp(jnp.maximum(t, 0.2 * t))
    wq = jnp.dot(wself, q_ref[...], preferred_element_type=jnp.float32)
    denom = jnp.dot(S, r_ref[...], preferred_element_type=jnp.float32) \
        + wq + 1e-16
    y = (msg + h * wq) / denom + b_ref[...]
    mu = jnp.mean(y, axis=-1, keepdims=True)
    var = jnp.mean((y - mu) ** 2, axis=-1, keepdims=True)
    out_ref[...] = (y - mu) * lax.rsqrt(var + 1e-5) * g_ref[...] + be_ref[...]


def _tc2(p0, p1, h, PS, Q, R, bias, gamma, beta):
    blk = 1000
    grid = N // blk
    full2 = lambda a, b: pl.BlockSpec((a, b), lambda i: (0, 0))
    return pl.pallas_call(
        _tc2_body,
        grid=(grid,),
        in_specs=[
            pl.BlockSpec((blk, ROW), lambda i: (i, 0)),
            pl.BlockSpec((blk, ROW), lambda i: (i, 0)),
            pl.BlockSpec((blk, D), lambda i: (i, 0)),
            full2(D, H),
            full2(H, D),
            full2(ROW, D),
            full2(1, D),
            full2(1, D),
            full2(1, D),
        ],
        out_specs=pl.BlockSpec((blk, D), lambda i: (i, 0)),
        out_shape=jax.ShapeDtypeStruct((N, D), jnp.float32),
    )(p0, p1, h, PS, Q, R, bias, gamma, beta)


def kernel(x, edge_index, W, att_src, att_dst, bias, ln_gamma, ln_beta):
    asf = att_src.reshape(-1)
    adf = att_dst.reshape(-1)
    eye = jnp.repeat(jnp.eye(H, dtype=jnp.float32), F_OUT, axis=0)
    Ps = eye * asf[:, None]
    Pd = eye * adf[:, None]
    P16 = jnp.concatenate([Ps, Pd], axis=1)
    PS = Ps + Pd
    Q = eye.T
    R = jnp.zeros((ROW, D), jnp.float32).at[D:D + H].set(Q)

    h, ad16 = _tc1(x, W, P16)
    zeros = jnp.zeros((N_PAD, ROW), jnp.float32)
    src_blocks = edge_index[0].reshape(NW * NB, B)
    dst_blocks = edge_index[1].reshape(NW * NB, B)
    il = jnp.concatenate([src_blocks, dst_blocks], axis=1)
    partials = _sc_edge_pass(il, h, ad16, zeros)
    out = _tc2(partials[0, :N], partials[1, :N], h, PS, Q, R,
               bias.reshape(1, D), ln_gamma.reshape(1, D),
               ln_beta.reshape(1, D))
    return out


# speedup vs baseline: 1.1237x; 1.1237x over previous
import functools

import jax
import jax.numpy as jnp
from jax import lax
from jax.experimental import pallas as pl
from jax.experimental.pallas import tpu as pltpu
from jax.experimental.pallas import tpu_sc as plsc

N = 10000
E = 320000
F_IN = 128
H = 8
F_OUT = 16
D = H * F_OUT
ROW = 144

NC = 2
NS = 16
NW = NC * NS
EPT = E // NW
B = 40
NB = EPT // B
N_PAD = 10240
RPT = N_PAD // NS


def _tc1_body(x_ref, w_ref, p_ref, h_ref, ad_ref):
    h = jnp.dot(x_ref[...], w_ref[...], preferred_element_type=jnp.float32)
    h_ref[...] = h
    ad_ref[...] = jnp.dot(h, p_ref[...], preferred_element_type=jnp.float32)


def _tc1(x, W, P16):
    blk = 1000
    grid = N // blk
    return pl.pallas_call(
        _tc1_body,
        grid=(grid,),
        in_specs=[
            pl.BlockSpec((blk, F_IN), lambda i: (i, 0)),
            pl.BlockSpec((F_IN, D), lambda i: (0, 0)),
            pl.BlockSpec((F_IN, 2 * H), lambda i: (0, 0)),
        ],
        out_specs=[
            pl.BlockSpec((blk, D), lambda i: (i, 0)),
            pl.BlockSpec((blk, 2 * H), lambda i: (i, 0)),
        ],
        out_shape=[
            jax.ShapeDtypeStruct((N, D), jnp.float32),
            jax.ShapeDtypeStruct((N, 2 * H), jnp.float32),
        ],
    )(x, W, P16)


def _sc_edge_pass(src, dst, h, ad16, zeros):
    mesh = plsc.VectorSubcoreMesh(core_axis_name="c", subcore_axis_name="s",
                                  num_cores=NC, num_subcores=NS)

    @functools.partial(
        pl.kernel,
        out_type=jax.ShapeDtypeStruct((NC, N_PAD, ROW), jnp.float32),
        mesh=mesh,
        scratch_types=[
            pltpu.VMEM((B,), jnp.int32),
            pltpu.VMEM((B,), jnp.int32),
            pltpu.VMEM((B,), jnp.int32),
            pltpu.VMEM((B,), jnp.int32),
            pltpu.VMEM((B,), jnp.int32),
            pltpu.VMEM((B,), jnp.int32),
            pltpu.VMEM((B, 2 * H), jnp.float32),
            pltpu.VMEM((B, 2 * H), jnp.float32),
            pltpu.VMEM((B, 2 * H), jnp.float32),
            pltpu.VMEM((B, 2 * H), jnp.float32),
            pltpu.VMEM((B, D), jnp.float32),
            pltpu.VMEM((B, D), jnp.float32),
            pltpu.VMEM((B, ROW), jnp.float32),
            pltpu.VMEM((B, ROW), jnp.float32),
            pltpu.VMEM_SHARED((N_PAD, ROW), jnp.float32),
            pltpu.SemaphoreType.DMA,
            pltpu.SemaphoreType.DMA,
            pltpu.SemaphoreType.DMA,
            pltpu.SemaphoreType.DMA,
            pltpu.SemaphoreType.DMA,
            pltpu.SemaphoreType.DMA,
            pltpu.SemaphoreType.DMA,
            pltpu.SemaphoreType.DMA,
        ],
        compiler_params=pltpu.CompilerParams(
            use_tc_tiling_on_sc=False, needs_layout_passes=False),
    )
    def sc_kernel(src_hbm, dst_hbm, h_hbm, ad_hbm, z_hbm, out_hbm,
                  sv0, sv1, dv0, dv1, dsc0, dsc1, g1_0, g1_1, g2_0, g2_1,
                  hr_0, hr_1, cb_0, cb_1, acc,
                  si0, si1, sd0, sd1, sg0, sg1, ss0, ss1):
        c = lax.axis_index("c")
        s = lax.axis_index("s")
        wid = c * NS + s
        bufs = ((sv0, dv0, dsc0, g1_0, g2_0, hr_0, cb_0, si0, sd0, sg0, ss0),
                (sv1, dv1, dsc1, g1_1, g2_1, hr_1, cb_1, si1, sd1, sg1, ss1))

        pltpu.sync_copy(z_hbm.at[pl.ds(s * RPT, RPT)],
                        acc.at[pl.ds(s * RPT, RPT)])
        zero16 = jnp.zeros((16,), jnp.float32)
        for e in range(B):
            cb_0[e, pl.ds(D, 16)] = zero16
            cb_1[e, pl.ds(D, 16)] = zero16
        plsc.subcore_barrier()

        base = wid * EPT
        iota = lax.iota(jnp.int32, 16)
        rowpat = iota // 8
        colpat = iota % 8

        def issue_idx(i, p):
            sv, dv = bufs[p][0], bufs[p][1]
            si = bufs[p][7]
            pltpu.async_copy(src_hbm.at[pl.ds(base + i * B, B)], sv, si)
            pltpu.async_copy(dst_hbm.at[pl.ds(base + i * B, B)], dv, si)

        def wait_idx(p):
            sv, dv = bufs[p][0], bufs[p][1]
            si = bufs[p][7]
            pltpu.make_async_copy(src_hbm.at[pl.ds(0, B)], sv, si).wait()
            pltpu.make_async_copy(dst_hbm.at[pl.ds(0, B)], dv, si).wait()

        def issue_g(p):
            sv, dv, _, g1, g2, hr = bufs[p][:6]
            sg = bufs[p][9]
            pltpu.async_copy(ad_hbm.at[sv], g1, sg)
            pltpu.async_copy(ad_hbm.at[dv], g2, sg)
            pltpu.async_copy(h_hbm.at[sv], hr, sg)

        def drain_g(p):
            g1, g2, hr = bufs[p][3], bufs[p][4], bufs[p][5]
            sg = bufs[p][9]
            pltpu.make_async_copy(ad_hbm.at[pl.ds(0, B)], g1, sg).wait()
            pltpu.make_async_copy(ad_hbm.at[pl.ds(0, B)], g2, sg).wait()
            pltpu.make_async_copy(h_hbm.at[pl.ds(0, B)], hr, sg).wait()

        def issue_dsc(i, p):
            dsc, sd = bufs[p][2], bufs[p][8]
            pltpu.async_copy(dst_hbm.at[pl.ds(base + i * B, B)], dsc, sd)

        def wait_dsc(p):
            dsc, sd = bufs[p][2], bufs[p][8]
            pltpu.make_async_copy(dst_hbm.at[pl.ds(0, B)], dsc, sd).wait()

        def drain_s(p):
            cb, ss = bufs[p][6], bufs[p][10]
            pltpu.make_async_copy(z_hbm.at[pl.ds(0, B)], cb, ss).wait()

        def compute(p):
            g1, g2, hr, cb = bufs[p][3], bufs[p][4], bufs[p][5], bufs[p][6]

            @plsc.parallel_loop(0, B // 2, 1, unroll=2)
            def pair(j):
                r = rowpat + 2 * j
                t = (plsc.load_gather(g1, [r, colpat])
                     + plsc.load_gather(g2, [r, colpat + 8]))
                w = jnp.exp(jnp.maximum(t, 0.2 * t))
                plsc.store_scatter(cb, [r, colpat + D], w)
                for k in range(16):
                    e = 2 * j + (k // 8)
                    col = (k % 8) * 16
                    bk = jnp.take_along_axis(
                        w, jnp.full((16,), k, jnp.int32), axis=0,
                        mode="promise_in_bounds")
                    cb[e, pl.ds(col, 16)] = hr[e, pl.ds(col, 16)] * bk

        def scatter(p):
            dsc, cb, ss = bufs[p][2], bufs[p][6], bufs[p][10]
            pltpu.async_copy(cb, acc.at[dsc], ss, add=True)

        issue_idx(0, 0)
        issue_idx(1, 1)
        wait_idx(0)
        issue_g(0)

        def pairbody(k, carry):
            for b in range(2):
                i = 2 * k + b

                def _nxt(p=b):
                    wait_idx(1 - p)
                    issue_g(1 - p)
                pl.when(i + 1 < NB)(_nxt)
                drain_g(b)

                def _pf(p=b, iv=i):
                    issue_idx(iv + 2, p)
                pl.when(i + 2 < NB)(_pf)

                pl.when(i >= 2)(lambda p=b: drain_s(p))
                issue_dsc(i, b)
                compute(b)
                wait_dsc(b)
                scatter(b)
            return carry

        lax.fori_loop(0, NB // 2, pairbody, 0)
        drain_s(0)
        drain_s(1)
        plsc.subcore_barrier()
        pltpu.sync_copy(acc.at[pl.ds(s * RPT, RPT)],
                        out_hbm.at[c, pl.ds(s * RPT, RPT)])

    return sc_kernel(src, dst, h, ad16, zeros)


def _tc2_body(p0_ref, p1_ref, h_ref, ps_ref, q_ref, r_ref, b_ref, g_ref,
              be_ref, out_ref):
    S = p0_ref[...] + p1_ref[...]
    h = h_ref[...]
    msg = S[:, :D]
    t = jnp.dot(h, ps_ref[...], preferred_element_type=jnp.float32)
    wself = jnp.exp(jnp.maximum(t, 0.2 * t))
    wq = jnp.dot(wself, q_ref[...], preferred_element_type=jnp.float32)
    denom = jnp.dot(S, r_ref[...], preferred_element_type=jnp.float32) \
        + wq + 1e-16
    y = (msg + h * wq) / denom + b_ref[...]
    mu = jnp.mean(y, axis=-1, keepdims=True)
    var = jnp.mean((y - mu) ** 2, axis=-1, keepdims=True)
    out_ref[...] = (y - mu) * lax.rsqrt(var + 1e-5) * g_ref[...] + be_ref[...]


def _tc2(p0, p1, h, PS, Q, R, bias, gamma, beta):
    blk = 1000
    grid = N // blk
    full2 = lambda a, b: pl.BlockSpec((a, b), lambda i: (0, 0))
    return pl.pallas_call(
        _tc2_body,
        grid=(grid,),
        in_specs=[
            pl.BlockSpec((blk, ROW), lambda i: (i, 0)),
            pl.BlockSpec((blk, ROW), lambda i: (i, 0)),
            pl.BlockSpec((blk, D), lambda i: (i, 0)),
            full2(D, H),
            full2(H, D),
            full2(ROW, D),
            full2(1, D),
            full2(1, D),
            full2(1, D),
        ],
        out_specs=pl.BlockSpec((blk, D), lambda i: (i, 0)),
        out_shape=jax.ShapeDtypeStruct((N, D), jnp.float32),
    )(p0, p1, h, PS, Q, R, bias, gamma, beta)


def kernel(x, edge_index, W, att_src, att_dst, bias, ln_gamma, ln_beta):
    asf = att_src.reshape(-1)
    adf = att_dst.reshape(-1)
    eye = jnp.repeat(jnp.eye(H, dtype=jnp.float32), F_OUT, axis=0)
    Ps = eye * asf[:, None]
    Pd = eye * adf[:, None]
    P16 = jnp.concatenate([Ps, Pd], axis=1)
    PS = Ps + Pd
    Q = eye.T
    R = jnp.zeros((ROW, D), jnp.float32).at[D:D + H].set(Q)

    h, ad16 = _tc1(x, W, P16)
    zeros = jnp.zeros((N_PAD, ROW), jnp.float32)
    partials = _sc_edge_pass(edge_index[0], edge_index[1], h, ad16, zeros)
    out = _tc2(partials[0, :N], partials[1, :N], h, PS, Q, R,
               bias.reshape(1, D), ln_gamma.reshape(1, D),
               ln_beta.reshape(1, D))
    return out
